# baseline (speedup 1.0000x reference)
import functools
import math

import jax
import jax.numpy as jnp
from jax import lax
from jax.experimental import pallas as pl
from jax.experimental.pallas import tpu as pltpu
from jax.experimental.pallas import tpu_sc as plsc

D_MODEL = 64
SCALE = math.sqrt(D_MODEL)
NUM_CORES = 2
NUM_SUBCORES = 16
NUM_WORKERS = NUM_CORES * NUM_SUBCORES
LANES = 16
NBUF = 2
CHUNK = 160


@jax.jit
def _gather_scale(x_flat, table2):
    n = x_flat.shape[0]
    per_worker = n // NUM_WORKERS
    n_chunks = per_worker // CHUNK
    rounds = n_chunks // NBUF
    half = CHUNK // 2
    mesh = plsc.VectorSubcoreMesh(core_axis_name="c", subcore_axis_name="s")

    @functools.partial(
        pl.kernel,
        out_type=jax.ShapeDtypeStruct((n // 2, 2 * D_MODEL), jnp.float32),
        mesh=mesh,
        scratch_types=[
            pltpu.VMEM((per_worker,), jnp.int32),
            pltpu.VMEM((per_worker,), jnp.int32),
            [pltpu.VMEM((CHUNK, 2 * D_MODEL), jnp.float32) for _ in range(NBUF)],
            [pltpu.VMEM((half, 2 * D_MODEL), jnp.float32) for _ in range(NBUF)],
            [pltpu.SemaphoreType.DMA for _ in range(NBUF)],
            [pltpu.SemaphoreType.DMA for _ in range(NBUF)],
        ],
    )
    def k(x_hbm, t2_hbm, out_hbm, idx_v, pair_v, in_bufs, out_bufs, g_sems, s_sems):
        wid = lax.axis_index("s") * NUM_CORES + lax.axis_index("c")
        base = wid * per_worker
        obase = wid * (per_worker // 2)

        pltpu.sync_copy(x_hbm.at[pl.ds(base, per_worker)], idx_v)

        @plsc.parallel_loop(0, per_worker // LANES, unroll=8)
        def _(i):
            pair_v[pl.ds(i * LANES, LANES)] = lax.shift_right_logical(
                idx_v[pl.ds(i * LANES, LANES)], 1
            )

        def gather_start(ci, b):
            pltpu.async_copy(
                t2_hbm.at[pair_v.at[pl.ds(ci * CHUNK, CHUNK)]],
                in_bufs[b],
                g_sems[b],
            )

        for b in range(NBUF):
            gather_start(b, b)

        def round_body(r, carry):
            for b in range(NBUF):
                ci = r * NBUF + b

                pltpu.make_async_copy(
                    t2_hbm.at[pair_v.at[pl.ds(ci * CHUNK, CHUNK)]],
                    in_bufs[b],
                    g_sems[b],
                ).wait()

                @pl.when(r > 0)
                def _():
                    pltpu.make_async_copy(
                        out_bufs[b],
                        out_hbm.at[pl.ds(obase + ci * half, half)],
                        s_sems[b],
                    ).wait()

                @plsc.parallel_loop(0, CHUNK // LANES, unroll=2)
                def _(jb):
                    svec = (idx_v[pl.ds(ci * CHUNK + jb * LANES, LANES)] & 1) * D_MODEL
                    for l in range(LANES):
                        par = svec[l]
                        j = jb * LANES + l
                        jo = jb * (LANES // 2) + l // 2
                        dst = (l & 1) * D_MODEL
                        for g in range(D_MODEL // LANES):
                            out_bufs[b][jo, pl.ds(dst + g * LANES, LANES)] = (
                                in_bufs[b][j, pl.ds(par + g * LANES, LANES)] * SCALE
                            )

                pltpu.async_copy(
                    out_bufs[b], out_hbm.at[pl.ds(obase + ci * half, half)], s_sems[b]
                )

                @pl.when(ci + NBUF < n_chunks)
                def _():
                    gather_start(ci + NBUF, b)

            return carry

        lax.fori_loop(0, rounds, round_body, 0)

        for b in range(NBUF):
            last = (rounds - 1) * NBUF + b
            pltpu.make_async_copy(
                out_bufs[b], out_hbm.at[pl.ds(obase + last * half, half)], s_sems[b]
            ).wait()

    return k(x_flat, table2)


def kernel(x, table):
    b, s = x.shape
    v, d = table.shape
    x_flat = x.reshape(b * s).astype(jnp.int32)
    table2 = table.reshape(v // 2, 2 * d)
    out = _gather_scale(x_flat, table2)
    return out.reshape(b, s, D_MODEL)


# speedup vs baseline: 1.1608x; 1.1608x over previous
import functools
import math

import jax
import jax.numpy as jnp
from jax import lax
from jax.experimental import pallas as pl
from jax.experimental.pallas import tpu as pltpu
from jax.experimental.pallas import tpu_sc as plsc

D_MODEL = 64
SCALE = math.sqrt(D_MODEL)
NUM_CORES = 2
NUM_SUBCORES = 16
NUM_WORKERS = NUM_CORES * NUM_SUBCORES
LANES = 16
VOCAB = 1000000
CH = 128
SEQ = 200
BATCH = 4096
HALF_B = BATCH // 2
NBUF = 2


PBLK = 1024
PGRID = (VOCAB + PBLK - 1) // PBLK
PROWS = PGRID * PBLK


def _prep_table(tt):
    def body(in_ref, out_ref):
        t = in_ref[...].T
        out_ref[:, 0:D_MODEL] = t
        out_ref[:, D_MODEL : 2 * D_MODEL] = t

    return pl.pallas_call(
        body,
        grid=(PGRID,),
        in_specs=[pl.BlockSpec((D_MODEL, PBLK), lambda i: (0, i))],
        out_specs=pl.BlockSpec((PBLK, 2 * D_MODEL), lambda i: (i, 0)),
        out_shape=jax.ShapeDtypeStruct((PROWS, 2 * D_MODEL), jnp.float32),
    )(tt)


def _finish_output(packed):
    def body(in_ref, out_ref):
        t = in_ref[...].T
        out_ref[0, :, 0:HALF_B] = t[0:D_MODEL, :]
        out_ref[0, :, HALF_B:BATCH] = t[D_MODEL : 2 * D_MODEL, :]

    return pl.pallas_call(
        body,
        grid=(SEQ,),
        in_specs=[pl.BlockSpec((HALF_B, 2 * D_MODEL), lambda s: (s, 0))],
        out_specs=pl.BlockSpec((1, D_MODEL, BATCH), lambda s: (s, 0, 0)),
        out_shape=jax.ShapeDtypeStruct((SEQ, D_MODEL, BATCH), jnp.float32),
    )(packed)


def _remap_idx(x2):
    def body(in_ref, out_ref):
        out_ref[...] = in_ref[...] * 2

    return pl.pallas_call(
        body,
        grid=(1,),
        in_specs=[pl.BlockSpec(x2.shape, lambda i: (0, 0))],
        out_specs=pl.BlockSpec(x2.shape, lambda i: (0, 0)),
        out_shape=jax.ShapeDtypeStruct(x2.shape, jnp.int32),
    )(x2)


def _gather_scale(xflat, tflat):
    n = xflat.shape[0]
    mesh = plsc.VectorSubcoreMesh(core_axis_name="c", subcore_axis_name="s")
    n_chunks = HALF_B // CH

    @functools.partial(
        pl.kernel,
        out_type=jax.ShapeDtypeStruct((n // 2, 2 * D_MODEL), jnp.float32),
        mesh=mesh,
        scratch_types=[
            [pltpu.VMEM((HALF_B,), jnp.int32) for _ in range(2)],
            [pltpu.VMEM((CH, D_MODEL), jnp.float32) for _ in range(2 * NBUF)],
            [pltpu.VMEM((CH, 2 * D_MODEL), jnp.float32) for _ in range(NBUF)],
            [pltpu.SemaphoreType.DMA for _ in range(2 * NBUF)],
            [pltpu.SemaphoreType.DMA for _ in range(NBUF)],
        ],
        compiler_params=pltpu.CompilerParams(use_tc_tiling_on_sc=False),
    )
    def k(x_hbm, t_hbm, out_hbm, idx, in_bufs, out_bufs, g_sems, s_sems):
        wid = lax.axis_index("s") * NUM_CORES + lax.axis_index("c")
        s_cnt = jnp.where(wid < 8, 7, 6)
        s0 = 6 * wid + jnp.minimum(wid, 8)

        def do_seq(si, carry):
            s = s0 + si
            for h in range(2):
                pltpu.sync_copy(
                    x_hbm.at[pl.ds(s * BATCH + h * HALF_B, HALF_B)], idx[h]
                )

            def gather_start(ci, b):
                for h in range(2):
                    pltpu.async_copy(
                        t_hbm.at[idx[h].at[pl.ds(ci * CH, CH)]],
                        in_bufs[2 * b + h],
                        g_sems[2 * b + h],
                    )

            for b in range(NBUF):
                gather_start(b, b)

            obase = s * HALF_B

            def chunk_body(ci, b):
                for h in range(2):
                    pltpu.make_async_copy(
                        t_hbm.at[idx[h].at[pl.ds(ci * CH, CH)]],
                        in_bufs[2 * b + h],
                        g_sems[2 * b + h],
                    ).wait()

                @pl.when(ci >= NBUF)
                def _():
                    pltpu.make_async_copy(
                        out_bufs[b],
                        out_hbm.at[pl.ds(obase + ci * CH, CH)],
                        s_sems[b],
                    ).wait()

                @plsc.parallel_loop(0, CH, unroll=8)
                def _(r):
                    for h in range(2):
                        for g in range(D_MODEL // LANES):
                            out_bufs[b][
                                r, pl.ds(h * D_MODEL + g * LANES, LANES)
                            ] = in_bufs[2 * b + h][r, pl.ds(g * LANES, LANES)] * SCALE

                pltpu.async_copy(
                    out_bufs[b], out_hbm.at[pl.ds(obase + ci * CH, CH)], s_sems[b]
                )

                @pl.when(ci + NBUF < n_chunks)
                def _():
                    gather_start(ci + NBUF, b)

            def round_body(r, c2):
                for b in range(NBUF):
                    chunk_body(r * NBUF + b, b)
                return c2

            lax.fori_loop(0, n_chunks // NBUF, round_body, 0)

            for b in range(NBUF):
                last = n_chunks - NBUF + b
                pltpu.make_async_copy(
                    out_bufs[b], out_hbm.at[pl.ds(obase + last * CH, CH)], s_sems[b]
                ).wait()

            return carry

        lax.fori_loop(0, s_cnt, do_seq, 0)

    return k(xflat, tflat)


@jax.jit
def _run(x, table):
    tt = table.T
    x2 = x.T.reshape(SEQ * BATCH // 128, 128)
    xflat = _remap_idx(x2.astype(jnp.int32)).reshape(SEQ * BATCH)
    t2 = _prep_table(tt)
    tflat = t2.reshape(2 * PROWS, D_MODEL)
    packed = _gather_scale(xflat, tflat)
    o3 = _finish_output(packed)
    return o3.transpose(2, 0, 1)


def kernel(x, table):
    return _run(x, table)


# speedup vs baseline: 1.2493x; 1.0762x over previous
import functools
import math

import jax
import jax.numpy as jnp
from jax import lax
from jax.experimental import pallas as pl
from jax.experimental.pallas import tpu as pltpu
from jax.experimental.pallas import tpu_sc as plsc

D_MODEL = 64
SCALE = math.sqrt(D_MODEL)
NUM_CORES = 2
NUM_SUBCORES = 16
NUM_WORKERS = NUM_CORES * NUM_SUBCORES
LANES = 16
VOCAB = 1000000
CH = 128
SEQ = 200
BATCH = 4096
HALF_B = BATCH // 2
NBUF = 2


PBLK = 512
PGRID = (VOCAB + 2 * PBLK - 1) // (2 * PBLK)
PROWS = PGRID * PBLK


def _prep_table(tt):
    def body(a_ref, b_ref, out_ref):
        out_ref[:, 0:D_MODEL] = a_ref[...].T
        out_ref[:, D_MODEL : 2 * D_MODEL] = b_ref[...].T

    return pl.pallas_call(
        body,
        grid=(PGRID,),
        in_specs=[
            pl.BlockSpec((D_MODEL, PBLK), lambda i: (0, i)),
            pl.BlockSpec((D_MODEL, PBLK), lambda i: (0, i + PGRID)),
        ],
        out_specs=pl.BlockSpec((PBLK, 2 * D_MODEL), lambda i: (i, 0)),
        out_shape=jax.ShapeDtypeStruct((PROWS, 2 * D_MODEL), jnp.float32),
    )(tt, tt)


def _finish_output(packed):
    def body(in_ref, out_ref):
        t = in_ref[...].T
        out_ref[0, :, 0:HALF_B] = t[0:D_MODEL, :]
        out_ref[0, :, HALF_B:BATCH] = t[D_MODEL : 2 * D_MODEL, :]

    return pl.pallas_call(
        body,
        grid=(SEQ,),
        in_specs=[pl.BlockSpec((HALF_B, 2 * D_MODEL), lambda s: (s, 0))],
        out_specs=pl.BlockSpec((1, D_MODEL, BATCH), lambda s: (s, 0, 0)),
        out_shape=jax.ShapeDtypeStruct((SEQ, D_MODEL, BATCH), jnp.float32),
    )(packed)


def _remap_idx(x2):
    def body(in_ref, out_ref):
        j = in_ref[...]
        out_ref[...] = 2 * j - jnp.where(j >= PROWS, 2 * PROWS - 1, 0)

    return pl.pallas_call(
        body,
        grid=(1,),
        in_specs=[pl.BlockSpec(x2.shape, lambda i: (0, 0))],
        out_specs=pl.BlockSpec(x2.shape, lambda i: (0, 0)),
        out_shape=jax.ShapeDtypeStruct(x2.shape, jnp.int32),
    )(x2)


def _gather_scale(xflat, tflat):
    n = xflat.shape[0]
    mesh = plsc.VectorSubcoreMesh(core_axis_name="c", subcore_axis_name="s")
    n_chunks = HALF_B // CH

    @functools.partial(
        pl.kernel,
        out_type=jax.ShapeDtypeStruct((n // 2, 2 * D_MODEL), jnp.float32),
        mesh=mesh,
        scratch_types=[
            [pltpu.VMEM((HALF_B,), jnp.int32) for _ in range(2)],
            [pltpu.VMEM((CH, D_MODEL), jnp.float32) for _ in range(2 * NBUF)],
            [pltpu.VMEM((CH, 2 * D_MODEL), jnp.float32) for _ in range(NBUF)],
            [pltpu.SemaphoreType.DMA for _ in range(2 * NBUF)],
            [pltpu.SemaphoreType.DMA for _ in range(NBUF)],
        ],
        compiler_params=pltpu.CompilerParams(use_tc_tiling_on_sc=False),
    )
    def k(x_hbm, t_hbm, out_hbm, idx, in_bufs, out_bufs, g_sems, s_sems):
        wid = lax.axis_index("s") * NUM_CORES + lax.axis_index("c")
        s_cnt = jnp.where(wid < 8, 7, 6)
        s0 = 6 * wid + jnp.minimum(wid, 8)

        def do_seq(si, carry):
            s = s0 + si
            for h in range(2):
                pltpu.sync_copy(
                    x_hbm.at[pl.ds(s * BATCH + h * HALF_B, HALF_B)], idx[h]
                )

            def gather_start(ci, b):
                for h in range(2):
                    pltpu.async_copy(
                        t_hbm.at[idx[h].at[pl.ds(ci * CH, CH)]],
                        in_bufs[2 * b + h],
                        g_sems[2 * b + h],
                    )

            for b in range(NBUF):
                gather_start(b, b)

            obase = s * HALF_B

            def chunk_body(ci, b):
                for h in range(2):
                    pltpu.make_async_copy(
                        t_hbm.at[idx[h].at[pl.ds(ci * CH, CH)]],
                        in_bufs[2 * b + h],
                        g_sems[2 * b + h],
                    ).wait()

                @pl.when(ci >= NBUF)
                def _():
                    pltpu.make_async_copy(
                        out_bufs[b],
                        out_hbm.at[pl.ds(obase + ci * CH, CH)],
                        s_sems[b],
                    ).wait()

                @plsc.parallel_loop(0, CH, unroll=8)
                def _(r):
                    for h in range(2):
                        for g in range(D_MODEL // LANES):
                            out_bufs[b][
                                r, pl.ds(h * D_MODEL + g * LANES, LANES)
                            ] = in_bufs[2 * b + h][r, pl.ds(g * LANES, LANES)] * SCALE

                pltpu.async_copy(
                    out_bufs[b], out_hbm.at[pl.ds(obase + ci * CH, CH)], s_sems[b]
                )

                @pl.when(ci + NBUF < n_chunks)
                def _():
                    gather_start(ci + NBUF, b)

            def round_body(r, c2):
                for b in range(NBUF):
                    chunk_body(r * NBUF + b, b)
                return c2

            lax.fori_loop(0, n_chunks // NBUF, round_body, 0)

            for b in range(NBUF):
                last = n_chunks - NBUF + b
                pltpu.make_async_copy(
                    out_bufs[b], out_hbm.at[pl.ds(obase + last * CH, CH)], s_sems[b]
                ).wait()

            return carry

        lax.fori_loop(0, s_cnt, do_seq, 0)

    return k(xflat, tflat)


@jax.jit
def _run(x, table):
    tt = table.T
    x2 = x.T.reshape(SEQ * BATCH // 128, 128)
    xflat = _remap_idx(x2.astype(jnp.int32)).reshape(SEQ * BATCH)
    t2 = _prep_table(tt)
    tflat = t2.reshape(2 * PROWS, D_MODEL)
    packed = _gather_scale(xflat, tflat)
    o3 = _finish_output(packed)
    return o3.transpose(2, 0, 1)


def kernel(x, table):
    return _run(x, table)


# speedup vs baseline: 1.7121x; 1.3705x over previous
import functools
import math

import jax
import jax.numpy as jnp
from jax import lax
from jax.experimental import pallas as pl
from jax.experimental.pallas import tpu as pltpu
from jax.experimental.pallas import tpu_sc as plsc

D_MODEL = 64
SCALE = math.sqrt(D_MODEL)
NUM_CORES = 2
NUM_SUBCORES = 16
NUM_WORKERS = NUM_CORES * NUM_SUBCORES
LANES = 16
VOCAB = 1000000
CH = 128
SEQ = 200
BATCH = 4096
HALF_B = BATCH // 2
NBUF = 2


PBLK = 512
PGRID = (VOCAB + 2 * PBLK - 1) // (2 * PBLK)
PROWS = PGRID * PBLK


def _prep_table(tt):
    def body(a1_ref, a2_ref, b1_ref, b2_ref, out_ref):
        out_ref[0:PBLK, 0:D_MODEL] = a1_ref[...].T
        out_ref[PBLK : 2 * PBLK, 0:D_MODEL] = a2_ref[...].T
        out_ref[0:PBLK, D_MODEL : 2 * D_MODEL] = b1_ref[...].T
        out_ref[PBLK : 2 * PBLK, D_MODEL : 2 * D_MODEL] = b2_ref[...].T

    return pl.pallas_call(
        body,
        grid=(PGRID // 2 + 1,),
        in_specs=[
            pl.BlockSpec((D_MODEL, PBLK), lambda i: (0, jnp.minimum(2 * i, PGRID - 1))),
            pl.BlockSpec((D_MODEL, PBLK), lambda i: (0, jnp.minimum(2 * i + 1, PGRID - 1))),
            pl.BlockSpec((D_MODEL, PBLK), lambda i: (0, jnp.minimum(2 * i + PGRID, 2 * PGRID - 1))),
            pl.BlockSpec((D_MODEL, PBLK), lambda i: (0, jnp.minimum(2 * i + 1 + PGRID, 2 * PGRID - 1))),
        ],
        out_specs=pl.BlockSpec((2 * PBLK, 2 * D_MODEL), lambda i: (i, 0)),
        out_shape=jax.ShapeDtypeStruct((PROWS, 2 * D_MODEL), jnp.float32),
    )(tt, tt, tt, tt)


def _finish_output(packed):
    def body(in_ref, out_ref):
        t = in_ref[...].T
        for q in range(2):
            c0, c1 = q * HALF_B, (q + 1) * HALF_B
            out_ref[q, :, 0:HALF_B] = t[0:D_MODEL, c0:c1]
            out_ref[q, :, HALF_B:BATCH] = t[D_MODEL : 2 * D_MODEL, c0:c1]

    return pl.pallas_call(
        body,
        grid=(SEQ // 2,),
        in_specs=[pl.BlockSpec((2 * HALF_B, 2 * D_MODEL), lambda s: (s, 0))],
        out_specs=pl.BlockSpec((2, D_MODEL, BATCH), lambda s: (s, 0, 0)),
        out_shape=jax.ShapeDtypeStruct((SEQ, D_MODEL, BATCH), jnp.float32),
    )(packed)


def _remap_idx(x2):
    def body(in_ref, out_ref):
        j = in_ref[...]
        out_ref[...] = 2 * j - jnp.where(j >= PROWS, 2 * PROWS - 1, 0)

    return pl.pallas_call(
        body,
        grid=(1,),
        in_specs=[pl.BlockSpec(x2.shape, lambda i: (0, 0))],
        out_specs=pl.BlockSpec(x2.shape, lambda i: (0, 0)),
        out_shape=jax.ShapeDtypeStruct(x2.shape, jnp.int32),
    )(x2)


def _gather_scale(xflat, tflat):
    n = xflat.shape[0]
    mesh = plsc.VectorSubcoreMesh(core_axis_name="c", subcore_axis_name="s")
    n_chunks = HALF_B // CH

    @functools.partial(
        pl.kernel,
        out_type=jax.ShapeDtypeStruct((n // 2, 2 * D_MODEL), jnp.float32),
        mesh=mesh,
        scratch_types=[
            [pltpu.VMEM((HALF_B,), jnp.int32) for _ in range(2)],
            [pltpu.VMEM((CH, D_MODEL), jnp.float32) for _ in range(2 * NBUF)],
            [pltpu.VMEM((CH, 2 * D_MODEL), jnp.float32) for _ in range(NBUF)],
            [pltpu.SemaphoreType.DMA for _ in range(2 * NBUF)],
            [pltpu.SemaphoreType.DMA for _ in range(NBUF)],
        ],
        compiler_params=pltpu.CompilerParams(use_tc_tiling_on_sc=False),
    )
    def k(x_hbm, t_hbm, out_hbm, idx, in_bufs, out_bufs, g_sems, s_sems):
        wid = lax.axis_index("s") * NUM_CORES + lax.axis_index("c")
        s_cnt = jnp.where(wid < 8, 7, 6)
        s0 = 6 * wid + jnp.minimum(wid, 8)

        def do_seq(si, carry):
            s = s0 + si
            for h in range(2):
                pltpu.sync_copy(
                    x_hbm.at[pl.ds(s * BATCH + h * HALF_B, HALF_B)], idx[h]
                )

            def gather_start(ci, b):
                for h in range(2):
                    pltpu.async_copy(
                        t_hbm.at[idx[h].at[pl.ds(ci * CH, CH)]],
                        in_bufs[2 * b + h],
                        g_sems[2 * b + h],
                    )

            for b in range(NBUF):
                gather_start(b, b)

            obase = s * HALF_B

            def chunk_body(ci, b):
                for h in range(2):
                    pltpu.make_async_copy(
                        t_hbm.at[idx[h].at[pl.ds(ci * CH, CH)]],
                        in_bufs[2 * b + h],
                        g_sems[2 * b + h],
                    ).wait()

                @pl.when(ci >= NBUF)
                def _():
                    pltpu.make_async_copy(
                        out_bufs[b],
                        out_hbm.at[pl.ds(obase + ci * CH, CH)],
                        s_sems[b],
                    ).wait()

                @plsc.parallel_loop(0, CH, unroll=8)
                def _(r):
                    for h in range(2):
                        for g in range(D_MODEL // LANES):
                            out_bufs[b][
                                r, pl.ds(h * D_MODEL + g * LANES, LANES)
                            ] = in_bufs[2 * b + h][r, pl.ds(g * LANES, LANES)] * SCALE

                pltpu.async_copy(
                    out_bufs[b], out_hbm.at[pl.ds(obase + ci * CH, CH)], s_sems[b]
                )

                @pl.when(ci + NBUF < n_chunks)
                def _():
                    gather_start(ci + NBUF, b)

            def round_body(r, c2):
                for b in range(NBUF):
                    chunk_body(r * NBUF + b, b)
                return c2

            lax.fori_loop(0, n_chunks // NBUF, round_body, 0)

            for b in range(NBUF):
                last = n_chunks - NBUF + b
                pltpu.make_async_copy(
                    out_bufs[b], out_hbm.at[pl.ds(obase + last * CH, CH)], s_sems[b]
                ).wait()

            return carry

        lax.fori_loop(0, s_cnt, do_seq, 0)

    return k(xflat, tflat)


@jax.jit
def _run(x, table):
    tt = table.T
    x2 = x.T.reshape(SEQ * BATCH // 128, 128)
    xflat = _remap_idx(x2.astype(jnp.int32)).reshape(SEQ * BATCH)
    t2 = _prep_table(tt)
    tflat = t2.reshape(2 * PROWS, D_MODEL)
    packed = _gather_scale(xflat, tflat)
    o3 = _finish_output(packed)
    return o3.transpose(2, 0, 1)


def kernel(x, table):
    return _run(x, table)


# speedup vs baseline: 2.0830x; 1.2166x over previous
import functools
import math

import jax
import jax.numpy as jnp
from jax import lax
from jax.experimental import pallas as pl
from jax.experimental.pallas import tpu as pltpu
from jax.experimental.pallas import tpu_sc as plsc

D_MODEL = 64
SCALE = math.sqrt(D_MODEL)
NUM_CORES = 2
NUM_SUBCORES = 16
NUM_WORKERS = NUM_CORES * NUM_SUBCORES
LANES = 16
VOCAB = 1000000
CH = 128
SEQ = 200
BATCH = 4096
HALF_B = BATCH // 2
NBUF = 2


PBLK = 512
PGRID = (VOCAB + 2 * PBLK - 1) // (2 * PBLK)
PROWS = PGRID * PBLK


def _prep_table(tt):
    NCH = 4

    def body(*refs):
        out_ref = refs[-1]
        for k in range(NCH):
            rows = slice(k * PBLK, (k + 1) * PBLK)
            out_ref[rows, 0:D_MODEL] = refs[k][...].T
            out_ref[rows, D_MODEL : 2 * D_MODEL] = refs[NCH + k][...].T

    def a_spec(k):
        return pl.BlockSpec(
            (D_MODEL, PBLK), lambda i, k=k: (0, jnp.minimum(NCH * i + k, PGRID - 1))
        )

    def b_spec(k):
        return pl.BlockSpec(
            (D_MODEL, PBLK),
            lambda i, k=k: (0, jnp.minimum(NCH * i + k + PGRID, 2 * PGRID - 1)),
        )

    return pl.pallas_call(
        body,
        grid=((PGRID + NCH - 1) // NCH,),
        in_specs=[a_spec(k) for k in range(NCH)] + [b_spec(k) for k in range(NCH)],
        out_specs=pl.BlockSpec((NCH * PBLK, 2 * D_MODEL), lambda i: (i, 0)),
        out_shape=jax.ShapeDtypeStruct((PROWS, 2 * D_MODEL), jnp.float32),
    )(*([tt] * (2 * NCH)))


def _finish_output(packed):
    NR = 4

    def body(in_ref, out_ref):
        t = in_ref[...].T
        for q in range(NR):
            c0, c1 = q * HALF_B, (q + 1) * HALF_B
            out_ref[q, :, 0:HALF_B] = t[0:D_MODEL, c0:c1]
            out_ref[q, :, HALF_B:BATCH] = t[D_MODEL : 2 * D_MODEL, c0:c1]

    return pl.pallas_call(
        body,
        grid=(SEQ // NR,),
        in_specs=[pl.BlockSpec((NR * HALF_B, 2 * D_MODEL), lambda s: (s, 0))],
        out_specs=pl.BlockSpec((NR, D_MODEL, BATCH), lambda s: (s, 0, 0)),
        out_shape=jax.ShapeDtypeStruct((SEQ, D_MODEL, BATCH), jnp.float32),
    )(packed)


def _remap_idx(x2):
    def body(in_ref, out_ref):
        j = in_ref[...]
        out_ref[...] = 2 * j - jnp.where(j >= PROWS, 2 * PROWS - 1, 0)

    return pl.pallas_call(
        body,
        grid=(1,),
        in_specs=[pl.BlockSpec(x2.shape, lambda i: (0, 0))],
        out_specs=pl.BlockSpec(x2.shape, lambda i: (0, 0)),
        out_shape=jax.ShapeDtypeStruct(x2.shape, jnp.int32),
    )(x2)


def _gather_scale(xflat, tflat):
    n = xflat.shape[0]
    mesh = plsc.VectorSubcoreMesh(core_axis_name="c", subcore_axis_name="s")
    n_chunks = HALF_B // CH

    @functools.partial(
        pl.kernel,
        out_type=jax.ShapeDtypeStruct((n // 2, 2 * D_MODEL), jnp.float32),
        mesh=mesh,
        scratch_types=[
            [pltpu.VMEM((HALF_B,), jnp.int32) for _ in range(2)],
            [pltpu.VMEM((CH, D_MODEL), jnp.float32) for _ in range(2 * NBUF)],
            [pltpu.VMEM((CH, 2 * D_MODEL), jnp.float32) for _ in range(NBUF)],
            [pltpu.SemaphoreType.DMA for _ in range(2 * NBUF)],
            [pltpu.SemaphoreType.DMA for _ in range(NBUF)],
        ],
        compiler_params=pltpu.CompilerParams(use_tc_tiling_on_sc=False),
    )
    def k(x_hbm, t_hbm, out_hbm, idx, in_bufs, out_bufs, g_sems, s_sems):
        wid = lax.axis_index("s") * NUM_CORES + lax.axis_index("c")
        s_cnt = jnp.where(wid < 8, 7, 6)
        s0 = 6 * wid + jnp.minimum(wid, 8)

        def do_seq(si, carry):
            s = s0 + si
            for h in range(2):
                pltpu.sync_copy(
                    x_hbm.at[pl.ds(s * BATCH + h * HALF_B, HALF_B)], idx[h]
                )

            def gather_start(ci, b):
                for h in range(2):
                    pltpu.async_copy(
                        t_hbm.at[idx[h].at[pl.ds(ci * CH, CH)]],
                        in_bufs[2 * b + h],
                        g_sems[2 * b + h],
                    )

            for b in range(NBUF):
                gather_start(b, b)

            obase = s * HALF_B

            def chunk_body(ci, b):
                for h in range(2):
                    pltpu.make_async_copy(
                        t_hbm.at[idx[h].at[pl.ds(ci * CH, CH)]],
                        in_bufs[2 * b + h],
                        g_sems[2 * b + h],
                    ).wait()

                @pl.when(ci >= NBUF)
                def _():
                    pltpu.make_async_copy(
                        out_bufs[b],
                        out_hbm.at[pl.ds(obase + ci * CH, CH)],
                        s_sems[b],
                    ).wait()

                @plsc.parallel_loop(0, CH, unroll=8)
                def _(r):
                    for h in range(2):
                        for g in range(D_MODEL // LANES):
                            out_bufs[b][
                                r, pl.ds(h * D_MODEL + g * LANES, LANES)
                            ] = in_bufs[2 * b + h][r, pl.ds(g * LANES, LANES)] * SCALE

                pltpu.async_copy(
                    out_bufs[b], out_hbm.at[pl.ds(obase + ci * CH, CH)], s_sems[b]
                )

                @pl.when(ci + NBUF < n_chunks)
                def _():
                    gather_start(ci + NBUF, b)

            def round_body(r, c2):
                for b in range(NBUF):
                    chunk_body(r * NBUF + b, b)
                return c2

            lax.fori_loop(0, n_chunks // NBUF, round_body, 0)

            for b in range(NBUF):
                last = n_chunks - NBUF + b
                pltpu.make_async_copy(
                    out_bufs[b], out_hbm.at[pl.ds(obase + last * CH, CH)], s_sems[b]
                ).wait()

            return carry

        lax.fori_loop(0, s_cnt, do_seq, 0)

    return k(xflat, tflat)


@jax.jit
def _run(x, table):
    tt = table.T
    x2 = x.T.reshape(SEQ * BATCH // 128, 128)
    xflat = _remap_idx(x2.astype(jnp.int32)).reshape(SEQ * BATCH)
    t2 = _prep_table(tt)
    tflat = t2.reshape(2 * PROWS, D_MODEL)
    packed = _gather_scale(xflat, tflat)
    o3 = _finish_output(packed)
    return o3.transpose(2, 0, 1)


def kernel(x, table):
    return _run(x, table)


# speedup vs baseline: 2.3115x; 1.1097x over previous
import functools
import math

import jax
import jax.numpy as jnp
from jax import lax
from jax.experimental import pallas as pl
from jax.experimental.pallas import tpu as pltpu
from jax.experimental.pallas import tpu_sc as plsc

D_MODEL = 64
SCALE = math.sqrt(D_MODEL)
NUM_CORES = 2
NUM_SUBCORES = 16
NUM_WORKERS = NUM_CORES * NUM_SUBCORES
LANES = 16
VOCAB = 1000000
CH = 128
SEQ = 200
BATCH = 4096
HALF_B = BATCH // 2
NBUF = 2


PBLK = 512
PGRID = (VOCAB + 2 * PBLK - 1) // (2 * PBLK)
PROWS = PGRID * PBLK


def _prep_table(tt):
    NCH = 8

    def body(*refs):
        out_ref = refs[-1]
        for k in range(NCH):
            rows = slice(k * PBLK, (k + 1) * PBLK)
            out_ref[rows, 0:D_MODEL] = refs[k][...].T
            out_ref[rows, D_MODEL : 2 * D_MODEL] = refs[NCH + k][...].T

    def a_spec(k):
        return pl.BlockSpec(
            (D_MODEL, PBLK), lambda i, k=k: (0, jnp.minimum(NCH * i + k, PGRID - 1))
        )

    def b_spec(k):
        return pl.BlockSpec(
            (D_MODEL, PBLK),
            lambda i, k=k: (0, jnp.minimum(NCH * i + k + PGRID, 2 * PGRID - 1)),
        )

    return pl.pallas_call(
        body,
        grid=((PGRID + NCH - 1) // NCH,),
        in_specs=[a_spec(k) for k in range(NCH)] + [b_spec(k) for k in range(NCH)],
        out_specs=pl.BlockSpec((NCH * PBLK, 2 * D_MODEL), lambda i: (i, 0)),
        out_shape=jax.ShapeDtypeStruct((PROWS, 2 * D_MODEL), jnp.float32),
    )(*([tt] * (2 * NCH)))


def _finish_output(packed):
    NR = 8

    def body(in_ref, out_ref):
        t = in_ref[...].T
        for q in range(NR):
            c0, c1 = q * HALF_B, (q + 1) * HALF_B
            out_ref[q, :, 0:HALF_B] = t[0:D_MODEL, c0:c1]
            out_ref[q, :, HALF_B:BATCH] = t[D_MODEL : 2 * D_MODEL, c0:c1]

    return pl.pallas_call(
        body,
        grid=(SEQ // NR,),
        in_specs=[pl.BlockSpec((NR * HALF_B, 2 * D_MODEL), lambda s: (s, 0))],
        out_specs=pl.BlockSpec((NR, D_MODEL, BATCH), lambda s: (s, 0, 0)),
        out_shape=jax.ShapeDtypeStruct((SEQ, D_MODEL, BATCH), jnp.float32),
    )(packed)


def _remap_idx(x2):
    def body(in_ref, out_ref):
        j = in_ref[...]
        out_ref[...] = 2 * j - jnp.where(j >= PROWS, 2 * PROWS - 1, 0)

    return pl.pallas_call(
        body,
        grid=(1,),
        in_specs=[pl.BlockSpec(x2.shape, lambda i: (0, 0))],
        out_specs=pl.BlockSpec(x2.shape, lambda i: (0, 0)),
        out_shape=jax.ShapeDtypeStruct(x2.shape, jnp.int32),
    )(x2)


def _gather_scale(xflat, tflat):
    n = xflat.shape[0]
    mesh = plsc.VectorSubcoreMesh(core_axis_name="c", subcore_axis_name="s")
    n_chunks = HALF_B // CH

    @functools.partial(
        pl.kernel,
        out_type=jax.ShapeDtypeStruct((n // 2, 2 * D_MODEL), jnp.float32),
        mesh=mesh,
        scratch_types=[
            [pltpu.VMEM((HALF_B,), jnp.int32) for _ in range(2)],
            [pltpu.VMEM((CH, D_MODEL), jnp.float32) for _ in range(2 * NBUF)],
            [pltpu.VMEM((CH, 2 * D_MODEL), jnp.float32) for _ in range(NBUF)],
            [pltpu.SemaphoreType.DMA for _ in range(2 * NBUF)],
            [pltpu.SemaphoreType.DMA for _ in range(NBUF)],
        ],
        compiler_params=pltpu.CompilerParams(use_tc_tiling_on_sc=False),
    )
    def k(x_hbm, t_hbm, out_hbm, idx, in_bufs, out_bufs, g_sems, s_sems):
        wid = lax.axis_index("s") * NUM_CORES + lax.axis_index("c")
        s_cnt = jnp.where(wid < 8, 7, 6)
        s0 = 6 * wid + jnp.minimum(wid, 8)

        def do_seq(si, carry):
            s = s0 + si
            for h in range(2):
                pltpu.sync_copy(
                    x_hbm.at[pl.ds(s * BATCH + h * HALF_B, HALF_B)], idx[h]
                )

            def gather_start(ci, b):
                for h in range(2):
                    pltpu.async_copy(
                        t_hbm.at[idx[h].at[pl.ds(ci * CH, CH)]],
                        in_bufs[2 * b + h],
                        g_sems[2 * b + h],
                    )

            for b in range(NBUF):
                gather_start(b, b)

            obase = s * HALF_B

            def chunk_body(ci, b):
                for h in range(2):
                    pltpu.make_async_copy(
                        t_hbm.at[idx[h].at[pl.ds(ci * CH, CH)]],
                        in_bufs[2 * b + h],
                        g_sems[2 * b + h],
                    ).wait()

                @pl.when(ci >= NBUF)
                def _():
                    pltpu.make_async_copy(
                        out_bufs[b],
                        out_hbm.at[pl.ds(obase + ci * CH, CH)],
                        s_sems[b],
                    ).wait()

                @plsc.parallel_loop(0, CH, unroll=8)
                def _(r):
                    for h in range(2):
                        for g in range(D_MODEL // LANES):
                            out_bufs[b][
                                r, pl.ds(h * D_MODEL + g * LANES, LANES)
                            ] = in_bufs[2 * b + h][r, pl.ds(g * LANES, LANES)] * SCALE

                pltpu.async_copy(
                    out_bufs[b], out_hbm.at[pl.ds(obase + ci * CH, CH)], s_sems[b]
                )

                @pl.when(ci + NBUF < n_chunks)
                def _():
                    gather_start(ci + NBUF, b)

            def round_body(r, c2):
                for b in range(NBUF):
                    chunk_body(r * NBUF + b, b)
                return c2

            lax.fori_loop(0, n_chunks // NBUF, round_body, 0)

            for b in range(NBUF):
                last = n_chunks - NBUF + b
                pltpu.make_async_copy(
                    out_bufs[b], out_hbm.at[pl.ds(obase + last * CH, CH)], s_sems[b]
                ).wait()

            return carry

        lax.fori_loop(0, s_cnt, do_seq, 0)

    return k(xflat, tflat)


@jax.jit
def _run(x, table):
    tt = table.T
    x2 = x.T.reshape(SEQ * BATCH // 128, 128)
    xflat = _remap_idx(x2.astype(jnp.int32)).reshape(SEQ * BATCH)
    t2 = _prep_table(tt)
    tflat = t2.reshape(2 * PROWS, D_MODEL)
    packed = _gather_scale(xflat, tflat)
    o3 = _finish_output(packed)
    return o3.transpose(2, 0, 1)


def kernel(x, table):
    return _run(x, table)


# speedup vs baseline: 2.3172x; 1.0025x over previous
import functools
import math

import jax
import jax.numpy as jnp
from jax import lax
from jax.experimental import pallas as pl
from jax.experimental.pallas import tpu as pltpu
from jax.experimental.pallas import tpu_sc as plsc

D_MODEL = 64
SCALE = math.sqrt(D_MODEL)
NUM_CORES = 2
NUM_SUBCORES = 16
NUM_WORKERS = NUM_CORES * NUM_SUBCORES
LANES = 16
VOCAB = 1000000
CH = 128
SEQ = 200
BATCH = 4096
HALF_B = BATCH // 2
NBUF = 2


PBLK = 512
PGRID = (VOCAB + 2 * PBLK - 1) // (2 * PBLK)
PROWS = PGRID * PBLK


def _prep_table(tt):
    NCH = 16

    def body(*refs):
        out_ref = refs[-1]
        for k in range(NCH):
            rows = slice(k * PBLK, (k + 1) * PBLK)
            out_ref[rows, 0:D_MODEL] = refs[k][...].T
            out_ref[rows, D_MODEL : 2 * D_MODEL] = refs[NCH + k][...].T

    def a_spec(k):
        return pl.BlockSpec(
            (D_MODEL, PBLK), lambda i, k=k: (0, jnp.minimum(NCH * i + k, PGRID - 1))
        )

    def b_spec(k):
        return pl.BlockSpec(
            (D_MODEL, PBLK),
            lambda i, k=k: (0, jnp.minimum(NCH * i + k + PGRID, 2 * PGRID - 1)),
        )

    return pl.pallas_call(
        body,
        grid=((PGRID + NCH - 1) // NCH,),
        in_specs=[a_spec(k) for k in range(NCH)] + [b_spec(k) for k in range(NCH)],
        out_specs=pl.BlockSpec((NCH * PBLK, 2 * D_MODEL), lambda i: (i, 0)),
        out_shape=jax.ShapeDtypeStruct((PROWS, 2 * D_MODEL), jnp.float32),
    )(*([tt] * (2 * NCH)))


def _finish_output(packed):
    NR = 10

    def body(in_ref, out_ref):
        t = in_ref[...].T
        for q in range(NR):
            c0, c1 = q * HALF_B, (q + 1) * HALF_B
            out_ref[q, :, 0:HALF_B] = t[0:D_MODEL, c0:c1]
            out_ref[q, :, HALF_B:BATCH] = t[D_MODEL : 2 * D_MODEL, c0:c1]

    return pl.pallas_call(
        body,
        grid=(SEQ // NR,),
        in_specs=[pl.BlockSpec((NR * HALF_B, 2 * D_MODEL), lambda s: (s, 0))],
        out_specs=pl.BlockSpec((NR, D_MODEL, BATCH), lambda s: (s, 0, 0)),
        out_shape=jax.ShapeDtypeStruct((SEQ, D_MODEL, BATCH), jnp.float32),
    )(packed)


def _remap_idx(x2):
    def body(in_ref, out_ref):
        j = in_ref[...]
        out_ref[...] = 2 * j - jnp.where(j >= PROWS, 2 * PROWS - 1, 0)

    return pl.pallas_call(
        body,
        grid=(1,),
        in_specs=[pl.BlockSpec(x2.shape, lambda i: (0, 0))],
        out_specs=pl.BlockSpec(x2.shape, lambda i: (0, 0)),
        out_shape=jax.ShapeDtypeStruct(x2.shape, jnp.int32),
    )(x2)


def _gather_scale(xflat, tflat):
    n = xflat.shape[0]
    mesh = plsc.VectorSubcoreMesh(core_axis_name="c", subcore_axis_name="s")
    n_chunks = HALF_B // CH

    @functools.partial(
        pl.kernel,
        out_type=jax.ShapeDtypeStruct((n // 2, 2 * D_MODEL), jnp.float32),
        mesh=mesh,
        scratch_types=[
            [pltpu.VMEM((HALF_B,), jnp.int32) for _ in range(2)],
            [pltpu.VMEM((CH, D_MODEL), jnp.float32) for _ in range(2 * NBUF)],
            [pltpu.VMEM((CH, 2 * D_MODEL), jnp.float32) for _ in range(NBUF)],
            [pltpu.SemaphoreType.DMA for _ in range(2 * NBUF)],
            [pltpu.SemaphoreType.DMA for _ in range(NBUF)],
        ],
        compiler_params=pltpu.CompilerParams(use_tc_tiling_on_sc=False),
    )
    def k(x_hbm, t_hbm, out_hbm, idx, in_bufs, out_bufs, g_sems, s_sems):
        wid = lax.axis_index("s") * NUM_CORES + lax.axis_index("c")
        s_cnt = jnp.where(wid < 8, 7, 6)
        s0 = 6 * wid + jnp.minimum(wid, 8)

        def do_seq(si, carry):
            s = s0 + si
            for h in range(2):
                pltpu.sync_copy(
                    x_hbm.at[pl.ds(s * BATCH + h * HALF_B, HALF_B)], idx[h]
                )

            def gather_start(ci, b):
                for h in range(2):
                    pltpu.async_copy(
                        t_hbm.at[idx[h].at[pl.ds(ci * CH, CH)]],
                        in_bufs[2 * b + h],
                        g_sems[2 * b + h],
                    )

            for b in range(NBUF):
                gather_start(b, b)

            obase = s * HALF_B

            def chunk_body(ci, b):
                for h in range(2):
                    pltpu.make_async_copy(
                        t_hbm.at[idx[h].at[pl.ds(ci * CH, CH)]],
                        in_bufs[2 * b + h],
                        g_sems[2 * b + h],
                    ).wait()

                @pl.when(ci >= NBUF)
                def _():
                    pltpu.make_async_copy(
                        out_bufs[b],
                        out_hbm.at[pl.ds(obase + ci * CH, CH)],
                        s_sems[b],
                    ).wait()

                @plsc.parallel_loop(0, CH, unroll=8)
                def _(r):
                    for h in range(2):
                        for g in range(D_MODEL // LANES):
                            out_bufs[b][
                                r, pl.ds(h * D_MODEL + g * LANES, LANES)
                            ] = in_bufs[2 * b + h][r, pl.ds(g * LANES, LANES)] * SCALE

                pltpu.async_copy(
                    out_bufs[b], out_hbm.at[pl.ds(obase + ci * CH, CH)], s_sems[b]
                )

                @pl.when(ci + NBUF < n_chunks)
                def _():
                    gather_start(ci + NBUF, b)

            def round_body(r, c2):
                for b in range(NBUF):
                    chunk_body(r * NBUF + b, b)
                return c2

            lax.fori_loop(0, n_chunks // NBUF, round_body, 0)

            for b in range(NBUF):
                last = n_chunks - NBUF + b
                pltpu.make_async_copy(
                    out_bufs[b], out_hbm.at[pl.ds(obase + last * CH, CH)], s_sems[b]
                ).wait()

            return carry

        lax.fori_loop(0, s_cnt, do_seq, 0)

    return k(xflat, tflat)


@jax.jit
def _run(x, table):
    tt = table.T
    x2 = x.T.reshape(SEQ * BATCH // 128, 128)
    xflat = _remap_idx(x2.astype(jnp.int32)).reshape(SEQ * BATCH)
    t2 = _prep_table(tt)
    tflat = t2.reshape(2 * PROWS, D_MODEL)
    packed = _gather_scale(xflat, tflat)
    o3 = _finish_output(packed)
    return o3.transpose(2, 0, 1)


def kernel(x, table):
    return _run(x, table)
